# baseline (speedup 1.0000x reference)
import jax
import jax.numpy as jnp
from jax import lax
from jax.experimental import pallas as pl
from jax.experimental.pallas import tpu as pltpu, tpu_sc as plsc

N = 10000
D = 128
E = 320000
ROUNDS = 3

NS = 16
CHUNK = 128
BLK = 8
NB = 20
N_CH = NB * BLK
E_PAD = NS * N_CH * CHUNK
N_ACC = 10112
ZR = N_ACC // NS
NFC = N // CHUNK
TAIL = N - NFC * CHUNK


def _body(x, ef, er, zacc, ones_h, zcnt,
          o0, o1, o2, o3, o4, o5,
          eblk0, eblk1, rows0, rows1, ones_v,
          accum_sh, cnt_sh, semg0, semg1, semb0, semb1):
    cid = lax.axis_index("c")
    sid = lax.axis_index("s")
    eblk = [eblk0, eblk1]
    rows = [rows0, rows1]
    semg = [semg0, semg1]
    semb = [semb0, semb1]

    def scale_rows(buf, cbuf, nrows):
        def fin_body(rr, carry):
            cnt = cbuf[rr, :]
            inv = jnp.float32(1.0) / jnp.maximum(cnt, jnp.float32(1.0))
            for j in range(D // 16):
                buf[rr, pl.ds(j * 16, 16)] = buf[rr, pl.ds(j * 16, 16)] * inv
            return carry
        lax.fori_loop(0, nrows, fin_body, 0)

    def run(e_hbm, outs):
        pltpu.sync_copy(ones_h, ones_v)
        h = x
        for r in range(ROUNDS):
            pltpu.sync_copy(zacc, accum_sh.at[pl.ds(sid * ZR, ZR)])
            if r == 0:
                pltpu.sync_copy(zcnt, cnt_sh.at[pl.ds(sid * ZR, ZR)])
            plsc.subcore_barrier()

            pltpu.make_async_copy(e_hbm.at[sid, 0], eblk0, semb0).start()
            pltpu.make_async_copy(e_hbm.at[sid, 1], eblk1, semb1).start()
            pltpu.make_async_copy(e_hbm.at[sid, 0], eblk0, semb0).wait()
            pltpu.make_async_copy(
                h.at[eblk0.at[0, 0]], rows0, semg0).start()

            def block_pair(p, carry):
                for bb in range(2):
                    k = 2 * p + bb
                    nbb = 1 - bb
                    for i in range(BLK):
                        j = i % 2
                        nj = 1 - j
                        if i < BLK - 1:
                            pltpu.make_async_copy(
                                h.at[eblk[bb].at[i + 1, 0]],
                                rows[nj], semg[nj]).start()
                        else:
                            @pl.when(k + 1 < NB)
                            def _():
                                pltpu.make_async_copy(
                                    e_hbm.at[sid, k + 1],
                                    eblk[nbb], semb[nbb]).wait()
                                pltpu.make_async_copy(
                                    h.at[eblk[nbb].at[0, 0]],
                                    rows[nj], semg[nj]).start()
                        pltpu.make_async_copy(
                            h.at[eblk[bb].at[i, 0]], rows[j], semg[j]).wait()
                        pltpu.sync_copy(
                            rows[j], accum_sh.at[eblk[bb].at[i, 1]], add=True)
                        if r == 0:
                            pltpu.sync_copy(
                                ones_v, cnt_sh.at[eblk[bb].at[i, 1]], add=True)

                    @pl.when(k + 2 < NB)
                    def _():
                        pltpu.make_async_copy(
                            e_hbm.at[sid, k + 2], eblk[bb], semb[bb]).start()
                return carry
            lax.fori_loop(0, NB // 2, block_pair, 0)
            plsc.subcore_barrier()

            for k in range(NFC // NS + 1):
                fc = sid + NS * k

                @pl.when(fc < NFC)
                def _():
                    c0 = fc * CHUNK
                    pltpu.sync_copy(accum_sh.at[pl.ds(c0, CHUNK)], rows0)
                    pltpu.sync_copy(cnt_sh.at[pl.ds(c0, CHUNK)], ones_v)
                    scale_rows(rows0, ones_v, CHUNK)
                    pltpu.sync_copy(rows0, outs[r].at[pl.ds(c0, CHUNK)])

            @pl.when(sid == NS - 1)
            def _():
                c0 = NFC * CHUNK
                pltpu.sync_copy(accum_sh.at[pl.ds(c0, TAIL)],
                                rows1.at[pl.ds(0, TAIL)])
                pltpu.sync_copy(cnt_sh.at[pl.ds(c0, TAIL)],
                                ones_v.at[pl.ds(0, TAIL)])
                scale_rows(rows1, ones_v, TAIL)
                pltpu.sync_copy(rows1.at[pl.ds(0, TAIL)],
                                outs[r].at[pl.ds(c0, TAIL)])

            plsc.subcore_barrier()
            h = outs[r]
            if r == 0:
                pltpu.sync_copy(ones_h, ones_v)

    @pl.when(cid == 0)
    def _():
        run(ef, [o0, o1, o2])

    @pl.when(cid == 1)
    def _():
        run(er, [o3, o4, o5])


@jax.jit
def kernel(topic_entity_one_hot, edge_index, reverse_edge_index):
    x = topic_entity_one_hot

    def prep(ei):
        pad_src = jnp.zeros((E_PAD - E,), jnp.int32)
        pad_dst = jnp.full((E_PAD - E,), N, jnp.int32)
        src = jnp.concatenate([ei[0], pad_src]).reshape(NS, N_CH, 1, CHUNK)
        dst = jnp.concatenate([ei[1], pad_dst]).reshape(NS, N_CH, 1, CHUNK)
        return jnp.concatenate([src, dst], axis=2).reshape(
            NS, NB, BLK, 2, CHUNK)

    ef = prep(edge_index)
    er = prep(reverse_edge_index)
    zacc = jnp.zeros((ZR, D), jnp.float32)
    ones = jnp.ones((CHUNK, 16), jnp.float32)
    zcnt = jnp.zeros((ZR, 16), jnp.float32)

    out = jax.ShapeDtypeStruct((N, D), jnp.float32)
    mesh = plsc.VectorSubcoreMesh(core_axis_name="c", subcore_axis_name="s")
    fn = pl.kernel(
        _body,
        out_type=(out,) * 6,
        mesh=mesh,
        compiler_params=pltpu.CompilerParams(use_tc_tiling_on_sc=False),
        scratch_types=[
            pltpu.VMEM((BLK, 2, CHUNK), jnp.int32),
            pltpu.VMEM((BLK, 2, CHUNK), jnp.int32),
            pltpu.VMEM((CHUNK, D), jnp.float32),
            pltpu.VMEM((CHUNK, D), jnp.float32),
            pltpu.VMEM((CHUNK, 16), jnp.float32),
            pltpu.VMEM_SHARED((N_ACC, D), jnp.float32),
            pltpu.VMEM_SHARED((N_ACC, 16), jnp.float32),
            pltpu.SemaphoreType.DMA,
            pltpu.SemaphoreType.DMA,
            pltpu.SemaphoreType.DMA,
            pltpu.SemaphoreType.DMA,
        ],
    )
    return fn(x, ef, er, zacc, ones, zcnt)
